# baseline (speedup 1.0000x reference)
import functools

import jax
import jax.numpy as jnp
from jax import lax
from jax.experimental import pallas as pl
from jax.experimental.pallas import tpu as pltpu
from jax.experimental.pallas import tpu_sc as plsc

D0, D1, D2 = 512, 512, 256
B = 16384
NC, NS, L = 2, 16, 16
NW = NC * NS
BPW = B // NW


def _gather_body(x0_hbm, x1_hbm, x2_hbm, board_hbm, out_hbm,
                 x0_v, x1_v, idx_v, val_v, sem):
    wid = lax.axis_index("s") * NC + lax.axis_index("c")
    base = wid * BPW
    c0 = pltpu.async_copy(x0_hbm.at[pl.ds(base, BPW)], x0_v, sem)
    c1 = pltpu.async_copy(x1_hbm.at[pl.ds(base, BPW)], x1_v, sem)
    c2 = pltpu.async_copy(x2_hbm.at[pl.ds(base, BPW)], idx_v, sem)
    c0.wait()
    c1.wait()
    c2.wait()
    for i in range(BPW // L):
        s = pl.ds(i * L, L)
        x1v = x1_v[s]
        x2v = idx_v[s]
        idx_v[s] = ((x0_v[s] << 17) + (x1v << 7) + ((x1v >> 3) << 10)
                    + x2v + (x2v >> 7) * 896)
    pltpu.async_copy(board_hbm.at[idx_v], val_v, sem).wait()
    pltpu.sync_copy(val_v, out_hbm.at[pl.ds(base, BPW)])


@jax.jit
def _gather(x0, x1, x2, board):
    mesh = plsc.VectorSubcoreMesh(core_axis_name="c", subcore_axis_name="s")
    fn = functools.partial(
        pl.kernel,
        mesh=mesh,
        out_type=jax.ShapeDtypeStruct((B,), jnp.float32),
        scratch_types=[
            pltpu.VMEM((BPW,), jnp.int32),
            pltpu.VMEM((BPW,), jnp.int32),
            pltpu.VMEM((BPW,), jnp.int32),
            pltpu.VMEM((BPW,), jnp.float32),
            pltpu.SemaphoreType.DMA,
        ],
        compiler_params=pltpu.CompilerParams(skip_device_barrier=True),
    )(_gather_body)
    return fn(x0, x1, x2, board)


def kernel(x0, x1, x2, board):
    x0 = x0.astype(jnp.int32)
    x1 = x1.astype(jnp.int32)
    x2 = x2.astype(jnp.int32)
    raw = (board.reshape(D0, D1 // 8, 8, D2 // 128, 128)
           .transpose(0, 1, 3, 2, 4)
           .reshape(D0 * D1 * D2))
    vals = _gather(x0, x1, x2, raw)
    return vals[:, None]


# speedup vs baseline: 1.0075x; 1.0075x over previous
import functools

import jax
import jax.numpy as jnp
from jax import lax
from jax.experimental import pallas as pl
from jax.experimental.pallas import tpu as pltpu
from jax.experimental.pallas import tpu_sc as plsc

D0, D1, D2 = 512, 512, 256
B = 16384
NC, NS, L = 2, 16, 16
NW = NC * NS
BPW = B // NW


NCHUNK = 2
CH = BPW // NCHUNK


def _gather_body(x0_hbm, x1_hbm, x2_hbm, board_hbm, out_hbm,
                 x0_v, x1_v, idx_v, val_v, sem, wsem):
    wid = lax.axis_index("s") * NC + lax.axis_index("c")
    base = wid * BPW
    c0 = pltpu.async_copy(x0_hbm.at[pl.ds(base, BPW)], x0_v, sem)
    c1 = pltpu.async_copy(x1_hbm.at[pl.ds(base, BPW)], x1_v, sem)
    c2 = pltpu.async_copy(x2_hbm.at[pl.ds(base, BPW)], idx_v, sem)
    c0.wait()
    c1.wait()
    c2.wait()
    gathers = []
    for c in range(NCHUNK):
        for i in range(c * CH // L, (c + 1) * CH // L):
            s = pl.ds(i * L, L)
            x1v = x1_v[s]
            x2v = idx_v[s]
            idx_v[s] = ((x0_v[s] << 17) + (x1v << 7) + ((x1v >> 3) << 10)
                        + x2v + (x2v >> 7) * 896)
        gathers.append(pltpu.async_copy(
            board_hbm.at[idx_v.at[pl.ds(c * CH, CH)]],
            val_v.at[pl.ds(c * CH, CH)], sem))
    writes = []
    for c in range(NCHUNK):
        gathers[c].wait()
        writes.append(pltpu.async_copy(
            val_v.at[pl.ds(c * CH, CH)],
            out_hbm.at[pl.ds(base + c * CH, CH)], wsem))
    for w in writes:
        w.wait()


@jax.jit
def _gather(x0, x1, x2, board):
    mesh = plsc.VectorSubcoreMesh(core_axis_name="c", subcore_axis_name="s")
    fn = functools.partial(
        pl.kernel,
        mesh=mesh,
        out_type=jax.ShapeDtypeStruct((B,), jnp.float32),
        scratch_types=[
            pltpu.VMEM((BPW,), jnp.int32),
            pltpu.VMEM((BPW,), jnp.int32),
            pltpu.VMEM((BPW,), jnp.int32),
            pltpu.VMEM((BPW,), jnp.float32),
            pltpu.SemaphoreType.DMA,
            pltpu.SemaphoreType.DMA,
        ],
    )(_gather_body)
    return fn(x0, x1, x2, board)


def kernel(x0, x1, x2, board):
    x0 = x0.astype(jnp.int32)
    x1 = x1.astype(jnp.int32)
    x2 = x2.astype(jnp.int32)
    raw = (board.reshape(D0, D1 // 8, 8, D2 // 128, 128)
           .transpose(0, 1, 3, 2, 4)
           .reshape(D0 * D1 * D2))
    vals = _gather(x0, x1, x2, raw)
    return vals[:, None]
